# baseline (speedup 1.0000x reference)
import functools
import math

import jax
import jax.numpy as jnp
from jax import lax
from jax.experimental import pallas as pl
from jax.experimental.pallas import tpu as pltpu
from jax.experimental.pallas import tpu_sc as plsc

D_MODEL = 64
SCALE = float(math.sqrt(D_MODEL))
BW = 128


@functools.lru_cache(maxsize=None)
def _make_embed(V, D, B, S):
    info = plsc.get_sparse_core_info()
    NC, NS, L = info.num_cores, info.num_subcores, info.num_lanes
    NW = NC * NS
    assert B == BW * NW and D == 64 and L == 16
    n_tok = BW * S
    mesh = plsc.VectorSubcoreMesh(core_axis_name="c", subcore_axis_name="s")

    @functools.partial(
        pl.kernel,
        out_type=jax.ShapeDtypeStruct((S, D // 8, B // BW, 8, BW), jnp.float32),
        mesh=mesh,
        scratch_types=(
            [pltpu.VMEM((n_tok,), jnp.int32)]
            + [pltpu.VMEM((BW,), jnp.int32) for _ in range(4)]
            + [pltpu.VMEM((BW, 2 * D), jnp.float32) for _ in range(2)]
            + [pltpu.VMEM((D // 8, 8, BW), jnp.float32) for _ in range(2)]
            + [pltpu.SemaphoreType.DMA for _ in range(4)]
        ),
        compiler_params=pltpu.CompilerParams(
            use_tc_tiling_on_sc=False, needs_layout_passes=False
        ),
    )
    def k(idx_hbm, tab2_hbm, out5_hbm, idx_all, *scr):
        idxp = scr[0:2]
        p64 = scr[2:4]
        rows = scr[4:6]
        obuf = scr[6:8]
        gsem = scr[8:10]
        ssem = scr[10:12]

        wid = lax.axis_index("s") * NC + lax.axis_index("c")
        pltpu.sync_copy(idx_hbm.at[pl.ds(wid * n_tok, n_tok)], idx_all)

        iota = jnp.arange(L, dtype=jnp.int32)
        iota_s = iota * S

        def prep(l, b):
            for g in range(BW // L):
                tix = iota_s + (g * L * S + l)
                v = plsc.load_gather(idx_all, [tix])
                idxp[b][pl.ds(g * L, L)] = v >> 1
                p64[b][pl.ds(g * L, L)] = (v & 1) << 6

        def gather_start(b):
            pltpu.async_copy(tab2_hbm.at[idxp[b]], rows[b], gsem[b])

        def gather_wait(b):
            pltpu.make_async_copy(tab2_hbm.at[idxp[b]], rows[b], gsem[b]).wait()

        def trans(b):
            for g in range(BW // L):
                rsel = iota + g * L
                pv = p64[b][pl.ds(g * L, L)]
                for d in range(D):
                    v = plsc.load_gather(rows[b], [rsel, pv + d])
                    obuf[b][d // 8, d % 8, pl.ds(g * L, L)] = v * SCALE

        def store_start(l, b):
            pltpu.async_copy(obuf[b], out5_hbm.at[l, :, wid], ssem[b])

        def store_wait(l, b):
            pltpu.make_async_copy(obuf[b], out5_hbm.at[l, :, wid], ssem[b]).wait()

        prep(0, 0)
        gather_start(0)

        @pl.loop(0, S, step=2)
        def _l0(l0):
            for b in range(2):
                l = l0 + b

                @pl.when(l + 1 < S)
                def _():
                    prep(l + 1, 1 - b)
                    gather_start(1 - b)

                gather_wait(b)

                @pl.when(l >= 2)
                def _():
                    store_wait(l, b)

                trans(b)
                store_start(l, b)

        store_wait(0, 0)
        store_wait(1, 1)

    return k


def kernel(x, table):
    B, S = x.shape
    V, D = table.shape
    idx = x.reshape(-1).astype(jnp.int32)
    tab2 = table.reshape(V // 2, 2 * D)
    out5 = _make_embed(V, D, B, S)(idx, tab2)
    o = out5.transpose(0, 1, 3, 2, 4)
    o = o.reshape(S, D, B)
    return o.transpose(2, 0, 1)


# speedup vs baseline: 1.6610x; 1.6610x over previous
import functools
import math

import jax
import jax.numpy as jnp
from jax import lax
from jax.experimental import pallas as pl
from jax.experimental.pallas import tpu as pltpu
from jax.experimental.pallas import tpu_sc as plsc

D_MODEL = 64
SCALE = float(math.sqrt(D_MODEL))
BW = 128


@functools.lru_cache(maxsize=None)
def _make_embed(V, D, B, S):
    info = plsc.get_sparse_core_info()
    NC, NS, L = info.num_cores, info.num_subcores, info.num_lanes
    NW = NC * NS
    assert B == BW * NW and D == 64 and L == 16
    n_tok = BW * S
    mesh = plsc.VectorSubcoreMesh(core_axis_name="c", subcore_axis_name="s")

    @functools.partial(
        pl.kernel,
        out_type=jax.ShapeDtypeStruct((S, D // 8, B // BW, 8, BW), jnp.float32),
        mesh=mesh,
        scratch_types=(
            [pltpu.VMEM((n_tok,), jnp.int32)]
            + [pltpu.VMEM((BW,), jnp.int32) for _ in range(4)]
            + [pltpu.VMEM((BW, 2 * D), jnp.float32) for _ in range(2)]
            + [pltpu.VMEM((D // 8, 8, BW), jnp.float32) for _ in range(2)]
            + [pltpu.SemaphoreType.DMA for _ in range(4)]
        ),
        compiler_params=pltpu.CompilerParams(
            use_tc_tiling_on_sc=False, needs_layout_passes=False
        ),
    )
    def k(idx_hbm, tab2_hbm, out5_hbm, idx_all, *scr):
        idxp = scr[0:2]
        p64 = scr[2:4]
        rows = scr[4:6]
        obuf = scr[6:8]
        gsem = scr[8:10]
        ssem = scr[10:12]

        wid = lax.axis_index("s") * NC + lax.axis_index("c")
        pltpu.sync_copy(idx_hbm.at[pl.ds(wid * n_tok, n_tok)], idx_all)

        iota = jnp.arange(L, dtype=jnp.int32)
        iota_s = iota * S

        def prep(l, b):
            @plsc.parallel_loop(0, BW, step=L, unroll=4)
            def _(t0):
                tix = iota_s + (t0 * S + l)
                v = plsc.load_gather(idx_all, [tix])
                idxp[b][pl.ds(t0, L)] = v >> 1
                p64[b][pl.ds(t0, L)] = (v & 1) << 6

        def gather_start(b):
            pltpu.async_copy(tab2_hbm.at[idxp[b]], rows[b], gsem[b])

        def gather_wait(b):
            pltpu.make_async_copy(tab2_hbm.at[idxp[b]], rows[b], gsem[b]).wait()

        def trans(b):
            rsels = [iota + g * L for g in range(BW // L)]
            pvs = [p64[b][pl.ds(g * L, L)] for g in range(BW // L)]

            @plsc.parallel_loop(0, D, unroll=8)
            def _(d):
                for g in range(BW // L):
                    v = plsc.load_gather(rows[b], [rsels[g], pvs[g] + d])
                    obuf[b][d >> 3, d & 7, pl.ds(g * L, L)] = v * SCALE

        def store_start(l, b):
            pltpu.async_copy(obuf[b], out5_hbm.at[l, :, wid], ssem[b])

        def store_wait(l, b):
            pltpu.make_async_copy(obuf[b], out5_hbm.at[l, :, wid], ssem[b]).wait()

        prep(0, 0)
        gather_start(0)

        @pl.loop(0, S, step=2)
        def _l0(l0):
            for b in range(2):
                l = l0 + b

                @pl.when(l + 1 < S)
                def _():
                    prep(l + 1, 1 - b)
                    gather_start(1 - b)

                gather_wait(b)

                @pl.when(l >= 2)
                def _():
                    store_wait(l, b)

                trans(b)
                store_start(l, b)

        store_wait(0, 0)
        store_wait(1, 1)

    return k


def kernel(x, table):
    B, S = x.shape
    V, D = table.shape
    idx = x.reshape(-1).astype(jnp.int32)
    tab2 = table.reshape(V // 2, 2 * D)
    out5 = _make_embed(V, D, B, S)(idx, tab2)
    o = out5.transpose(0, 1, 3, 2, 4)
    o = o.reshape(S, D, B)
    return o.transpose(2, 0, 1)
